# baseline (speedup 1.0000x reference)
import functools
import jax
import jax.numpy as jnp
from jax import lax
from jax.experimental import pallas as pl
from jax.experimental.pallas import tpu as pltpu
from jax.experimental.pallas import tpu_sc as plsc

D = 128
CHUNK = 128
NC = 2
NS = 16
NW = NC * NS


def _sc_mesh():
    return plsc.VectorSubcoreMesh(core_axis_name="c", subcore_axis_name="s")


def _make_deg_kernel(n_pad, e_per_w):
    rows_per_tec = n_pad // NS
    n_chunks = e_per_w // CHUNK

    @functools.partial(
        pl.kernel,
        mesh=_sc_mesh(),
        out_type=jax.ShapeDtypeStruct((NC, n_pad, D), jnp.float32),
        scratch_types=[
            pltpu.VMEM((CHUNK,), jnp.int32),
            pltpu.VMEM((CHUNK,), jnp.int32),
            pltpu.VMEM((CHUNK, D), jnp.float32),
            pltpu.VMEM((CHUNK, D), jnp.float32),
            pltpu.VMEM_SHARED((n_pad, D), jnp.float32),
        ],
    )
    def deg_kernel(src_hbm, dst_hbm, usrc_hbm, udst_hbm, zeros_hbm,
                   deg_out,
                   src_v, dst_v, usrc_v, udst_v, deg_sh):
        c = lax.axis_index("c")
        s = lax.axis_index("s")
        w = c * NS + s
        my_rows = pl.ds(s * rows_per_tec, rows_per_tec)
        pltpu.sync_copy(zeros_hbm, deg_sh.at[my_rows])
        pltpu.sync_copy(usrc_hbm, usrc_v)
        pltpu.sync_copy(udst_hbm, udst_v)
        plsc.subcore_barrier()

        def body(i, carry):
            base = pl.multiple_of(w * e_per_w + i * CHUNK, CHUNK)
            pltpu.sync_copy(src_hbm.at[pl.ds(base, CHUNK)], src_v)
            pltpu.sync_copy(dst_hbm.at[pl.ds(base, CHUNK)], dst_v)
            pltpu.sync_copy(usrc_v, deg_sh.at[src_v], add=True)
            pltpu.sync_copy(udst_v, deg_sh.at[dst_v], add=True)
            return carry

        lax.fori_loop(0, n_chunks, body, 0)
        plsc.subcore_barrier()
        pltpu.sync_copy(deg_sh.at[my_rows], deg_out.at[c, my_rows])

    return deg_kernel


def _make_edge_kernel(n_pad, e_per_w):
    rows_per_tec = n_pad // NS
    n_chunks = e_per_w // CHUNK
    NBUF = 2
    assert n_chunks % NBUF == 0

    @functools.partial(
        pl.kernel,
        mesh=_sc_mesh(),
        out_type=jax.ShapeDtypeStruct((NC, n_pad, D), jnp.float32),
        scratch_types=[
            pltpu.VMEM((n_chunks, CHUNK), jnp.int32),
        ] + [pltpu.VMEM((CHUNK,), jnp.int32)] * NBUF
          + [pltpu.VMEM((CHUNK, D), jnp.float32)] * NBUF + [
            pltpu.VMEM_SHARED((n_pad, D), jnp.float32),
        ] + [pltpu.SemaphoreType.DMA] * (3 * NBUF),
    )
    def edge_kernel(h_hbm, src_hbm, dst_hbm, zeros_hbm, agg_out,
                    src_v, *rest):
        didx = rest[:NBUF]
        bufs = rest[NBUF:2 * NBUF]
        agg_sh = rest[2 * NBUF]
        sems = rest[2 * NBUF + 1:]
        gsems = sems[:NBUF]
        ssems = sems[NBUF:2 * NBUF]
        dsems = sems[2 * NBUF:]
        c = lax.axis_index("c")
        s = lax.axis_index("s")
        w = c * NS + s
        my_rows = pl.ds(s * rows_per_tec, rows_per_tec)
        pltpu.sync_copy(zeros_hbm, agg_sh.at[my_rows])
        pltpu.sync_copy(src_hbm.at[pl.ds(w * n_chunks, n_chunks)], src_v)

        def start_didx(b, chunk):
            base = pl.multiple_of((w * n_chunks + chunk) * CHUNK, CHUNK)
            pltpu.async_copy(dst_hbm.at[pl.ds(base, CHUNK)], didx[b],
                             dsems[b])

        def wait_didx(b):
            pltpu.make_async_copy(dst_hbm.at[pl.ds(0, CHUNK)], didx[b],
                                  dsems[b]).wait()

        def start_gather(b, chunk):
            pltpu.async_copy(h_hbm.at[src_v.at[chunk]], bufs[b], gsems[b])

        def wait_gather(b):
            pltpu.make_async_copy(h_hbm.at[src_v.at[0]], bufs[b],
                                  gsems[b]).wait()

        def start_scatter(b):
            pltpu.async_copy(bufs[b], agg_sh.at[didx[b]], ssems[b], add=True)

        def wait_scatter(b):
            pltpu.make_async_copy(bufs[b], agg_sh.at[didx[b]],
                                  ssems[b]).wait()

        for b in range(NBUF):
            start_didx(b, b)
            start_gather(b, b)
        plsc.subcore_barrier()

        def body(j, carry):
            for b in range(NBUF):
                wait_gather(b)
                wait_didx(b)
                start_scatter(b)
            for b in range(NBUF):
                wait_scatter(b)
                nxt = j * NBUF + b + NBUF

                @pl.when(nxt < n_chunks)
                def _():
                    start_didx(b, nxt)
                    start_gather(b, nxt)
            return carry

        lax.fori_loop(0, n_chunks // NBUF, body, 0)
        plsc.subcore_barrier()
        pltpu.sync_copy(agg_sh.at[my_rows], agg_out.at[c, my_rows])

    return edge_kernel


def _norm_col(deg_ref):
    deg = deg_ref[0, :, 0:1] + deg_ref[1, :, 0:1]
    return lax.rsqrt(jnp.maximum(deg, 1.0))


def _mm_scale_body(x_ref, w_ref, dout_ref, o_ref):
    ns = _norm_col(dout_ref)
    o_ref[...] = jnp.dot(x_ref[...], w_ref[...],
                         preferred_element_type=jnp.float32) * ns


def _boundary_body(agg_ref, din_ref, dout_ref, b_ref, w_ref, o_ref):
    agg = agg_ref[0] + agg_ref[1]
    nd = _norm_col(din_ref)
    t = jnp.maximum(agg * nd + b_ref[...], 0.0)
    ns = _norm_col(dout_ref)
    o_ref[...] = jnp.dot(t, w_ref[...],
                         preferred_element_type=jnp.float32) * ns


def _final_body(agg_ref, din_ref, b_ref, o_ref):
    agg = agg_ref[0] + agg_ref[1]
    nd = _norm_col(din_ref)
    o_ref[...] = agg * nd + b_ref[...]


def _tc_grid_call(body, n_pad, r, ins, in_specs):
    grid = n_pad // r
    return pl.pallas_call(
        body,
        grid=(grid,),
        in_specs=in_specs,
        out_specs=pl.BlockSpec((r, D), lambda i: (i, 0)),
        out_shape=jax.ShapeDtypeStruct((n_pad, D), jnp.float32),
    )(*ins)


def _spec_rows(r):
    return pl.BlockSpec((r, D), lambda i: (i, 0))


def _spec_deg(r):
    return pl.BlockSpec((NC, r, 16), lambda i: (0, i, 0))


def _spec_agg(r):
    return pl.BlockSpec((NC, r, D), lambda i: (0, i, 0))


def _spec_full(shape):
    nd = len(shape)
    return pl.BlockSpec(shape, lambda i: (0,) * nd)


def kernel(x, edge_index, W1, b1, W2, b2, W3, b3):
    n = x.shape[0]
    e = edge_index.shape[1]

    r = 1024
    n_pad = ((n + 1 + r - 1) // r) * r
    grain = CHUNK * 4
    e_per_w = ((e + NW * grain - 1) // (NW * grain)) * grain
    e_pad = e_per_w * NW

    pad_idx = jnp.full((e_pad - e,), n, dtype=jnp.int32)
    src = jnp.concatenate([edge_index[0], pad_idx])
    dst = jnp.concatenate([edge_index[1], pad_idx])
    xp = jnp.pad(x, ((0, n_pad - n), (0, 0)))

    rows_per_tec = n_pad // NS
    zeros_rows = jnp.zeros((rows_per_tec, D), jnp.float32)
    half = D // 2
    col = jnp.arange(D)
    u_src = jnp.broadcast_to((col < half).astype(jnp.float32), (CHUNK, D))
    u_dst = jnp.broadcast_to((col >= half).astype(jnp.float32), (CHUNK, D))

    deg_kernel = _make_deg_kernel(n_pad, e_per_w)
    edge_kernel = _make_edge_kernel(n_pad, e_per_w)
    n_chunks = e_per_w // CHUNK
    src2 = src.reshape(NW * n_chunks, CHUNK)
    deg_tbl = deg_kernel(src, dst, u_src, u_dst, zeros_rows)
    dsrc = lax.slice(deg_tbl, (0, 0, 0), (NC, n_pad, 16))
    ddst = lax.slice(deg_tbl, (0, 0, half), (NC, n_pad, half + 16))

    b1r = b1.reshape(1, D)
    b2r = b2.reshape(1, D)
    b3r = b3.reshape(1, D)

    h1 = _tc_grid_call(
        _mm_scale_body, n_pad, r,
        [xp, W1, dsrc],
        [_spec_rows(r), _spec_full((D, D)), _spec_deg(r)],
    )
    a1 = edge_kernel(h1, src2, dst, zeros_rows)

    h2 = _tc_grid_call(
        _boundary_body, n_pad, r,
        [a1, ddst, dsrc, b1r, W2],
        [_spec_agg(r), _spec_deg(r), _spec_deg(r), _spec_full((1, D)),
         _spec_full((D, D))],
    )
    a2 = edge_kernel(h2, src2, dst, zeros_rows)

    h3 = _tc_grid_call(
        _boundary_body, n_pad, r,
        [a2, ddst, dsrc, b2r, W3],
        [_spec_agg(r), _spec_deg(r), _spec_deg(r), _spec_full((1, D)),
         _spec_full((D, D))],
    )
    a3 = edge_kernel(h3, src2, dst, zeros_rows)

    out = _tc_grid_call(
        _final_body, n_pad, r,
        [a3, ddst, b3r],
        [_spec_agg(r), _spec_deg(r), _spec_full((1, D))],
    )
    return out[:n]


# speedup vs baseline: 1.0176x; 1.0176x over previous
import functools
import jax
import jax.numpy as jnp
from jax import lax
from jax.experimental import pallas as pl
from jax.experimental.pallas import tpu as pltpu
from jax.experimental.pallas import tpu_sc as plsc

D = 128
CHUNK = 128
NC = 2
NS = 16
NW = NC * NS


def _sc_mesh():
    return plsc.VectorSubcoreMesh(core_axis_name="c", subcore_axis_name="s")


def _make_deg_kernel(n_pad, e_per_w):
    rows_per_tec = n_pad // NS
    n_chunks = e_per_w // CHUNK

    @functools.partial(
        pl.kernel,
        mesh=_sc_mesh(),
        out_type=jax.ShapeDtypeStruct((NC, n_pad, D), jnp.float32),
        scratch_types=[
            pltpu.VMEM((CHUNK,), jnp.int32),
            pltpu.VMEM((CHUNK,), jnp.int32),
            pltpu.VMEM((CHUNK, D), jnp.float32),
            pltpu.VMEM((CHUNK, D), jnp.float32),
            pltpu.VMEM_SHARED((n_pad, D), jnp.float32),
        ],
    )
    def deg_kernel(src_hbm, dst_hbm, usrc_hbm, udst_hbm, zeros_hbm,
                   deg_out,
                   src_v, dst_v, usrc_v, udst_v, deg_sh):
        c = lax.axis_index("c")
        s = lax.axis_index("s")
        w = c * NS + s
        my_rows = pl.ds(s * rows_per_tec, rows_per_tec)
        pltpu.sync_copy(zeros_hbm, deg_sh.at[my_rows])
        pltpu.sync_copy(usrc_hbm, usrc_v)
        pltpu.sync_copy(udst_hbm, udst_v)
        plsc.subcore_barrier()

        def body(i, carry):
            base = pl.multiple_of(w * e_per_w + i * CHUNK, CHUNK)
            pltpu.sync_copy(src_hbm.at[pl.ds(base, CHUNK)], src_v)
            pltpu.sync_copy(dst_hbm.at[pl.ds(base, CHUNK)], dst_v)
            pltpu.sync_copy(usrc_v, deg_sh.at[src_v], add=True)
            pltpu.sync_copy(udst_v, deg_sh.at[dst_v], add=True)
            return carry

        lax.fori_loop(0, n_chunks, body, 0)
        plsc.subcore_barrier()
        pltpu.sync_copy(deg_sh.at[my_rows], deg_out.at[c, my_rows])

    return deg_kernel


def _make_edge_kernel(n_pad, e_per_w):
    rows_per_tec = n_pad // NS
    n_chunks = e_per_w // CHUNK
    NBUF = 2
    assert n_chunks % NBUF == 0

    @functools.partial(
        pl.kernel,
        mesh=_sc_mesh(),
        out_type=jax.ShapeDtypeStruct((NC, n_pad, D), jnp.float32),
        scratch_types=[
            pltpu.VMEM((n_chunks, CHUNK), jnp.int32),
        ] + [pltpu.VMEM((CHUNK,), jnp.int32)] * NBUF
          + [pltpu.VMEM((CHUNK, D), jnp.float32)] * NBUF + [
            pltpu.VMEM_SHARED((n_pad, D), jnp.float32),
        ] + [pltpu.SemaphoreType.DMA] * (3 * NBUF),
    )
    def edge_kernel(h_hbm, src_hbm, dst_hbm, zeros_hbm, agg_out,
                    src_v, *rest):
        didx = rest[:NBUF]
        bufs = rest[NBUF:2 * NBUF]
        agg_sh = rest[2 * NBUF]
        sems = rest[2 * NBUF + 1:]
        gsems = sems[:NBUF]
        ssems = sems[NBUF:2 * NBUF]
        dsems = sems[2 * NBUF:]
        c = lax.axis_index("c")
        s = lax.axis_index("s")
        w = c * NS + s
        my_rows = pl.ds(s * rows_per_tec, rows_per_tec)
        pltpu.sync_copy(zeros_hbm, agg_sh.at[my_rows])
        pltpu.sync_copy(src_hbm.at[pl.ds(w * n_chunks, n_chunks)], src_v)

        def start_didx(b, chunk):
            base = pl.multiple_of((w * n_chunks + chunk) * CHUNK, CHUNK)
            pltpu.async_copy(dst_hbm.at[pl.ds(base, CHUNK)], didx[b],
                             dsems[b])

        def wait_didx(b):
            pltpu.make_async_copy(dst_hbm.at[pl.ds(0, CHUNK)], didx[b],
                                  dsems[b]).wait()

        def start_gather(b, chunk):
            pltpu.async_copy(h_hbm.at[src_v.at[chunk]], bufs[b], gsems[b])

        def wait_gather(b):
            pltpu.make_async_copy(h_hbm.at[src_v.at[0]], bufs[b],
                                  gsems[b]).wait()

        def start_scatter(b):
            pltpu.async_copy(bufs[b], agg_sh.at[didx[b]], ssems[b], add=True)

        def wait_scatter(b):
            pltpu.make_async_copy(bufs[b], agg_sh.at[didx[b]],
                                  ssems[b]).wait()

        for b in range(NBUF):
            start_didx(b, b)
            start_gather(b, b)
        plsc.subcore_barrier()

        def body(j, carry):
            for b in range(NBUF):
                wait_gather(b)
                wait_didx(b)
                pltpu.sync_copy(bufs[b], agg_sh.at[didx[b]], add=True)
                nxt = j * NBUF + b + NBUF

                @pl.when(nxt < n_chunks)
                def _():
                    start_didx(b, nxt)
                    start_gather(b, nxt)
            return carry

        lax.fori_loop(0, n_chunks // NBUF, body, 0)
        plsc.subcore_barrier()
        pltpu.sync_copy(agg_sh.at[my_rows], agg_out.at[c, my_rows])

    return edge_kernel


def _norm_col(deg_ref):
    deg = deg_ref[0, :, 0:1] + deg_ref[1, :, 0:1]
    return lax.rsqrt(jnp.maximum(deg, 1.0))


def _mm_scale_body(x_ref, w_ref, dout_ref, o_ref):
    ns = _norm_col(dout_ref)
    o_ref[...] = jnp.dot(x_ref[...], w_ref[...],
                         preferred_element_type=jnp.float32) * ns


def _boundary_body(agg_ref, din_ref, dout_ref, b_ref, w_ref, o_ref):
    agg = agg_ref[0] + agg_ref[1]
    nd = _norm_col(din_ref)
    t = jnp.maximum(agg * nd + b_ref[...], 0.0)
    ns = _norm_col(dout_ref)
    o_ref[...] = jnp.dot(t, w_ref[...],
                         preferred_element_type=jnp.float32) * ns


def _final_body(agg_ref, din_ref, b_ref, o_ref):
    agg = agg_ref[0] + agg_ref[1]
    nd = _norm_col(din_ref)
    o_ref[...] = agg * nd + b_ref[...]


def _tc_grid_call(body, n_pad, r, ins, in_specs):
    grid = n_pad // r
    return pl.pallas_call(
        body,
        grid=(grid,),
        in_specs=in_specs,
        out_specs=pl.BlockSpec((r, D), lambda i: (i, 0)),
        out_shape=jax.ShapeDtypeStruct((n_pad, D), jnp.float32),
    )(*ins)


def _spec_rows(r):
    return pl.BlockSpec((r, D), lambda i: (i, 0))


def _spec_deg(r):
    return pl.BlockSpec((NC, r, 16), lambda i: (0, i, 0))


def _spec_agg(r):
    return pl.BlockSpec((NC, r, D), lambda i: (0, i, 0))


def _spec_full(shape):
    nd = len(shape)
    return pl.BlockSpec(shape, lambda i: (0,) * nd)


def kernel(x, edge_index, W1, b1, W2, b2, W3, b3):
    n = x.shape[0]
    e = edge_index.shape[1]

    r = 1024
    n_pad = ((n + 1 + r - 1) // r) * r
    grain = CHUNK * 4
    e_per_w = ((e + NW * grain - 1) // (NW * grain)) * grain
    e_pad = e_per_w * NW

    pad_idx = jnp.full((e_pad - e,), n, dtype=jnp.int32)
    src = jnp.concatenate([edge_index[0], pad_idx])
    dst = jnp.concatenate([edge_index[1], pad_idx])
    xp = jnp.pad(x, ((0, n_pad - n), (0, 0)))

    rows_per_tec = n_pad // NS
    zeros_rows = jnp.zeros((rows_per_tec, D), jnp.float32)
    half = D // 2
    col = jnp.arange(D)
    u_src = jnp.broadcast_to((col < half).astype(jnp.float32), (CHUNK, D))
    u_dst = jnp.broadcast_to((col >= half).astype(jnp.float32), (CHUNK, D))

    deg_kernel = _make_deg_kernel(n_pad, e_per_w)
    edge_kernel = _make_edge_kernel(n_pad, e_per_w)
    n_chunks = e_per_w // CHUNK
    src2 = src.reshape(NW * n_chunks, CHUNK)
    deg_tbl = deg_kernel(src, dst, u_src, u_dst, zeros_rows)
    dsrc = lax.slice(deg_tbl, (0, 0, 0), (NC, n_pad, 16))
    ddst = lax.slice(deg_tbl, (0, 0, half), (NC, n_pad, half + 16))

    b1r = b1.reshape(1, D)
    b2r = b2.reshape(1, D)
    b3r = b3.reshape(1, D)

    h1 = _tc_grid_call(
        _mm_scale_body, n_pad, r,
        [xp, W1, dsrc],
        [_spec_rows(r), _spec_full((D, D)), _spec_deg(r)],
    )
    a1 = edge_kernel(h1, src2, dst, zeros_rows)

    h2 = _tc_grid_call(
        _boundary_body, n_pad, r,
        [a1, ddst, dsrc, b1r, W2],
        [_spec_agg(r), _spec_deg(r), _spec_deg(r), _spec_full((1, D)),
         _spec_full((D, D))],
    )
    a2 = edge_kernel(h2, src2, dst, zeros_rows)

    h3 = _tc_grid_call(
        _boundary_body, n_pad, r,
        [a2, ddst, dsrc, b2r, W3],
        [_spec_agg(r), _spec_deg(r), _spec_deg(r), _spec_full((1, D)),
         _spec_full((D, D))],
    )
    a3 = edge_kernel(h3, src2, dst, zeros_rows)

    out = _tc_grid_call(
        _final_body, n_pad, r,
        [a3, ddst, b3r],
        [_spec_agg(r), _spec_deg(r), _spec_full((1, D))],
    )
    return out[:n]


# speedup vs baseline: 1.4348x; 1.4100x over previous
import functools
import jax
import jax.numpy as jnp
from jax import lax
from jax.experimental import pallas as pl
from jax.experimental.pallas import tpu as pltpu
from jax.experimental.pallas import tpu_sc as plsc

D = 128
CHUNK = 128
NC = 2
NS = 16
NW = NC * NS


def _sc_mesh():
    return plsc.VectorSubcoreMesh(core_axis_name="c", subcore_axis_name="s")


def _make_deg_kernel(n_pad, e_per_w):
    rows_per_tec = n_pad // NS
    n_chunks = e_per_w // CHUNK

    @functools.partial(
        pl.kernel,
        mesh=_sc_mesh(),
        out_type=jax.ShapeDtypeStruct((NC, n_pad, D), jnp.float32),
        scratch_types=[
            pltpu.VMEM((CHUNK,), jnp.int32),
            pltpu.VMEM((CHUNK,), jnp.int32),
            pltpu.VMEM((CHUNK, D), jnp.float32),
            pltpu.VMEM((CHUNK, D), jnp.float32),
            pltpu.VMEM_SHARED((n_pad, D), jnp.float32),
        ],
    )
    def deg_kernel(src_hbm, dst_hbm, usrc_hbm, udst_hbm, zeros_hbm,
                   deg_out,
                   src_v, dst_v, usrc_v, udst_v, deg_sh):
        c = lax.axis_index("c")
        s = lax.axis_index("s")
        w = c * NS + s
        my_rows = pl.ds(s * rows_per_tec, rows_per_tec)
        pltpu.sync_copy(zeros_hbm, deg_sh.at[my_rows])
        pltpu.sync_copy(usrc_hbm, usrc_v)
        pltpu.sync_copy(udst_hbm, udst_v)
        plsc.subcore_barrier()

        def body(i, carry):
            base = pl.multiple_of(w * e_per_w + i * CHUNK, CHUNK)
            pltpu.sync_copy(src_hbm.at[pl.ds(base, CHUNK)], src_v)
            pltpu.sync_copy(dst_hbm.at[pl.ds(base, CHUNK)], dst_v)
            pltpu.sync_copy(usrc_v, deg_sh.at[src_v], add=True)
            pltpu.sync_copy(udst_v, deg_sh.at[dst_v], add=True)
            return carry

        lax.fori_loop(0, n_chunks, body, 0)
        plsc.subcore_barrier()
        pltpu.sync_copy(deg_sh.at[my_rows], deg_out.at[c, my_rows])

    return deg_kernel


def _make_edge_kernel(n_pad, nch0, nch1):
    rows_per_tec = n_pad // NS
    NBUF = 2
    assert nch0 % NBUF == 0 and nch1 % NBUF == 0

    @functools.partial(
        pl.kernel,
        mesh=_sc_mesh(),
        out_type=jax.ShapeDtypeStruct((NC, n_pad, D), jnp.float32),
        scratch_types=[pltpu.VMEM((CHUNK,), jnp.int32)] * (2 * NBUF)
          + [pltpu.VMEM((CHUNK, D), jnp.float32)] * NBUF + [
            pltpu.VMEM_SHARED((n_pad, D), jnp.float32),
        ] + [pltpu.SemaphoreType.DMA] * (3 * NBUF),
    )
    def edge_kernel(h_hbm, src_hbm, dst_hbm, zeros_hbm, agg_out, *rest):
        sidx = rest[:NBUF]
        didx = rest[NBUF:2 * NBUF]
        bufs = rest[2 * NBUF:3 * NBUF]
        agg_sh = rest[3 * NBUF]
        sems = rest[3 * NBUF + 1:]
        gsems = sems[:NBUF]
        isems = sems[NBUF:2 * NBUF]
        dsems = sems[2 * NBUF:]
        c = lax.axis_index("c")
        s = lax.axis_index("s")
        my_rows = pl.ds(s * rows_per_tec, rows_per_tec)
        nch = lax.select(c == 0, nch0, nch1)
        cb = lax.select(c == 0, s * nch0, NS * nch0 + s * nch1)
        pltpu.sync_copy(zeros_hbm, agg_sh.at[my_rows])

        def start_idx(b, chunk):
            base = pl.multiple_of((cb + chunk) * CHUNK, CHUNK)
            pltpu.async_copy(src_hbm.at[pl.ds(base, CHUNK)], sidx[b],
                             isems[b])
            pltpu.async_copy(dst_hbm.at[pl.ds(base, CHUNK)], didx[b],
                             dsems[b])

        def wait_sidx(b):
            pltpu.make_async_copy(src_hbm.at[pl.ds(0, CHUNK)], sidx[b],
                                  isems[b]).wait()

        def wait_didx(b):
            pltpu.make_async_copy(dst_hbm.at[pl.ds(0, CHUNK)], didx[b],
                                  dsems[b]).wait()

        def start_gather(b):
            pltpu.async_copy(h_hbm.at[sidx[b]], bufs[b], gsems[b])

        def wait_gather(b):
            pltpu.make_async_copy(h_hbm.at[sidx[b]], bufs[b],
                                  gsems[b]).wait()

        for b in range(NBUF):
            @pl.when(b < nch)
            def _():
                start_idx(b, b)
                wait_sidx(b)
                start_gather(b)
        plsc.subcore_barrier()

        def body(j, carry):
            for b in range(NBUF):
                cur = j * NBUF + b

                @pl.when(cur < nch)
                def _():
                    wait_gather(b)
                    wait_didx(b)
                    pltpu.sync_copy(bufs[b], agg_sh.at[didx[b]], add=True)
                    nxt = cur + NBUF

                    @pl.when(nxt < nch)
                    def _():
                        start_idx(b, nxt)
                        wait_sidx(b)
                        start_gather(b)
            return carry

        max_nch = max(nch0, nch1)
        lax.fori_loop(0, max_nch // NBUF, body, 0)
        plsc.subcore_barrier()
        pltpu.sync_copy(agg_sh.at[my_rows], agg_out.at[c, my_rows])

    return edge_kernel


def _norm_col(deg_ref):
    deg = deg_ref[0, :, 0:1] + deg_ref[1, :, 0:1]
    return lax.rsqrt(jnp.maximum(deg, 1.0))


def _mm_scale_body(x_ref, w_ref, dout_ref, o_ref):
    ns = _norm_col(dout_ref)
    o_ref[...] = jnp.dot(x_ref[...], w_ref[...],
                         preferred_element_type=jnp.float32) * ns


def _boundary_body(agg_ref, din_ref, dout_ref, b_ref, w_ref, o_ref):
    agg = agg_ref[0] + agg_ref[1]
    nd = _norm_col(din_ref)
    t = jnp.maximum(agg * nd + b_ref[...], 0.0)
    ns = _norm_col(dout_ref)
    o_ref[...] = jnp.dot(t, w_ref[...],
                         preferred_element_type=jnp.float32) * ns


def _final_body(agg_ref, din_ref, b_ref, o_ref):
    agg = agg_ref[0] + agg_ref[1]
    nd = _norm_col(din_ref)
    o_ref[...] = agg * nd + b_ref[...]


def _tc_grid_call(body, n_pad, r, ins, in_specs):
    grid = n_pad // r
    return pl.pallas_call(
        body,
        grid=(grid,),
        in_specs=in_specs,
        out_specs=pl.BlockSpec((r, D), lambda i: (i, 0)),
        out_shape=jax.ShapeDtypeStruct((n_pad, D), jnp.float32),
    )(*ins)


def _spec_rows(r):
    return pl.BlockSpec((r, D), lambda i: (i, 0))


def _spec_deg(r):
    return pl.BlockSpec((NC, r, 16), lambda i: (0, i, 0))


def _spec_agg(r):
    return pl.BlockSpec((NC, r, D), lambda i: (0, i, 0))


def _spec_full(shape):
    nd = len(shape)
    return pl.BlockSpec(shape, lambda i: (0,) * nd)


def kernel(x, edge_index, W1, b1, W2, b2, W3, b3):
    n = x.shape[0]
    e = edge_index.shape[1]

    r = 1024
    n_pad = ((n + 1 + r - 1) // r) * r
    t_chunks = -(-e // (NS * CHUNK))
    if t_chunks % 2:
        t_chunks += 1
    nch0 = max(2, 2 * int(round(t_chunks * 0.2 / 2)))
    nch1 = t_chunks - nch0
    e_pad = NS * t_chunks * CHUNK
    e_per_w = e_pad // NW

    pad_idx = jnp.full((e_pad - e,), n, dtype=jnp.int32)
    src = jnp.concatenate([edge_index[0], pad_idx])
    dst = jnp.concatenate([edge_index[1], pad_idx])
    xp = jnp.pad(x, ((0, n_pad - n), (0, 0)))

    rows_per_tec = n_pad // NS
    zeros_rows = jnp.zeros((rows_per_tec, D), jnp.float32)
    half = D // 2
    col = jnp.arange(D)
    u_src = jnp.broadcast_to((col < half).astype(jnp.float32), (CHUNK, D))
    u_dst = jnp.broadcast_to((col >= half).astype(jnp.float32), (CHUNK, D))

    deg_kernel = _make_deg_kernel(n_pad, e_per_w)
    edge_kernel = _make_edge_kernel(n_pad, nch0, nch1)
    deg_tbl = deg_kernel(src, dst, u_src, u_dst, zeros_rows)
    dsrc = lax.slice(deg_tbl, (0, 0, 0), (NC, n_pad, 16))
    ddst = lax.slice(deg_tbl, (0, 0, half), (NC, n_pad, half + 16))

    b1r = b1.reshape(1, D)
    b2r = b2.reshape(1, D)
    b3r = b3.reshape(1, D)

    h1 = _tc_grid_call(
        _mm_scale_body, n_pad, r,
        [xp, W1, dsrc],
        [_spec_rows(r), _spec_full((D, D)), _spec_deg(r)],
    )
    a1 = edge_kernel(h1, src, dst, zeros_rows)

    h2 = _tc_grid_call(
        _boundary_body, n_pad, r,
        [a1, ddst, dsrc, b1r, W2],
        [_spec_agg(r), _spec_deg(r), _spec_deg(r), _spec_full((1, D)),
         _spec_full((D, D))],
    )
    a2 = edge_kernel(h2, src, dst, zeros_rows)

    h3 = _tc_grid_call(
        _boundary_body, n_pad, r,
        [a2, ddst, dsrc, b2r, W3],
        [_spec_agg(r), _spec_deg(r), _spec_deg(r), _spec_full((1, D)),
         _spec_full((D, D))],
    )
    a3 = edge_kernel(h3, src, dst, zeros_rows)

    out = _tc_grid_call(
        _final_body, n_pad, r,
        [a3, ddst, b3r],
        [_spec_agg(r), _spec_deg(r), _spec_full((1, D))],
    )
    return out[:n]


# speedup vs baseline: 1.5589x; 1.0865x over previous
import functools
import jax
import jax.numpy as jnp
from jax import lax
from jax.experimental import pallas as pl
from jax.experimental.pallas import tpu as pltpu
from jax.experimental.pallas import tpu_sc as plsc

D = 128
CHUNK = 128
NC = 2
NS = 16
NW = NC * NS


def _sc_mesh():
    return plsc.VectorSubcoreMesh(core_axis_name="c", subcore_axis_name="s")


def _make_deg_kernel(n_pad, e_per_w):
    rows_per_tec = n_pad // NS
    n_chunks = e_per_w // CHUNK

    @functools.partial(
        pl.kernel,
        mesh=_sc_mesh(),
        out_type=jax.ShapeDtypeStruct((NC, n_pad, D), jnp.float32),
        scratch_types=[
            pltpu.VMEM((CHUNK,), jnp.int32),
            pltpu.VMEM((CHUNK,), jnp.int32),
            pltpu.VMEM((CHUNK, D), jnp.float32),
            pltpu.VMEM((CHUNK, D), jnp.float32),
            pltpu.VMEM_SHARED((n_pad, D), jnp.float32),
        ],
    )
    def deg_kernel(src_hbm, dst_hbm, usrc_hbm, udst_hbm, zeros_hbm,
                   deg_out,
                   src_v, dst_v, usrc_v, udst_v, deg_sh):
        c = lax.axis_index("c")
        s = lax.axis_index("s")
        w = c * NS + s
        my_rows = pl.ds(s * rows_per_tec, rows_per_tec)
        pltpu.sync_copy(zeros_hbm, deg_sh.at[my_rows])
        pltpu.sync_copy(usrc_hbm, usrc_v)
        pltpu.sync_copy(udst_hbm, udst_v)
        plsc.subcore_barrier()

        def body(i, carry):
            base = pl.multiple_of(w * e_per_w + i * CHUNK, CHUNK)
            pltpu.sync_copy(src_hbm.at[pl.ds(base, CHUNK)], src_v)
            pltpu.sync_copy(dst_hbm.at[pl.ds(base, CHUNK)], dst_v)
            pltpu.sync_copy(usrc_v, deg_sh.at[src_v], add=True)
            pltpu.sync_copy(udst_v, deg_sh.at[dst_v], add=True)
            return carry

        lax.fori_loop(0, n_chunks, body, 0)
        plsc.subcore_barrier()
        pltpu.sync_copy(deg_sh.at[my_rows], deg_out.at[c, my_rows])

    return deg_kernel


def _make_edge_kernel(n_pad, nch0, nch1):
    rows_per_tec = n_pad // NS
    NBUF = 2
    assert nch0 % NBUF == 0 and nch1 % NBUF == 0

    @functools.partial(
        pl.kernel,
        mesh=_sc_mesh(),
        out_type=jax.ShapeDtypeStruct((NC, n_pad, D), jnp.float32),
        scratch_types=[pltpu.VMEM((CHUNK,), jnp.int32)] * (2 * NBUF)
          + [pltpu.VMEM((CHUNK, D), jnp.float32)] * NBUF + [
            pltpu.VMEM_SHARED((n_pad, D), jnp.float32),
        ] + [pltpu.SemaphoreType.DMA] * (3 * NBUF),
    )
    def edge_kernel(h_hbm, src_hbm, dst_hbm, zeros_hbm, agg_out, *rest):
        sidx = rest[:NBUF]
        didx = rest[NBUF:2 * NBUF]
        bufs = rest[2 * NBUF:3 * NBUF]
        agg_sh = rest[3 * NBUF]
        sems = rest[3 * NBUF + 1:]
        gsems = sems[:NBUF]
        isems = sems[NBUF:2 * NBUF]
        dsems = sems[2 * NBUF:]
        c = lax.axis_index("c")
        s = lax.axis_index("s")
        my_rows = pl.ds(s * rows_per_tec, rows_per_tec)
        nch = lax.select(c == 0, nch0, nch1)
        cb = lax.select(c == 0, s * nch0, NS * nch0 + s * nch1)
        pltpu.sync_copy(zeros_hbm, agg_sh.at[my_rows])

        def start_idx(b, chunk):
            base = pl.multiple_of((cb + chunk) * CHUNK, CHUNK)
            pltpu.async_copy(src_hbm.at[pl.ds(base, CHUNK)], sidx[b],
                             isems[b])
            pltpu.async_copy(dst_hbm.at[pl.ds(base, CHUNK)], didx[b],
                             dsems[b])

        def wait_sidx(b):
            pltpu.make_async_copy(src_hbm.at[pl.ds(0, CHUNK)], sidx[b],
                                  isems[b]).wait()

        def wait_didx(b):
            pltpu.make_async_copy(dst_hbm.at[pl.ds(0, CHUNK)], didx[b],
                                  dsems[b]).wait()

        def start_gather(b):
            pltpu.async_copy(h_hbm.at[sidx[b]], bufs[b], gsems[b])

        def wait_gather(b):
            pltpu.make_async_copy(h_hbm.at[sidx[b]], bufs[b],
                                  gsems[b]).wait()

        for b in range(NBUF):
            @pl.when(b < nch)
            def _():
                start_idx(b, b)
                wait_sidx(b)
                start_gather(b)
        plsc.subcore_barrier()

        def body(j, carry):
            for b in range(NBUF):
                cur = j * NBUF + b

                @pl.when(cur < nch)
                def _():
                    wait_gather(b)
                    wait_didx(b)
                    pltpu.sync_copy(bufs[b], agg_sh.at[didx[b]], add=True)
                    nxt = cur + NBUF

                    @pl.when(nxt < nch)
                    def _():
                        start_idx(b, nxt)
                        wait_sidx(b)
                        start_gather(b)
            return carry

        max_nch = max(nch0, nch1)
        lax.fori_loop(0, max_nch // NBUF, body, 0)
        plsc.subcore_barrier()
        pltpu.sync_copy(agg_sh.at[my_rows], agg_out.at[c, my_rows])

    return edge_kernel


def _norm_col(deg_ref):
    deg = deg_ref[0, :, 0:1] + deg_ref[1, :, 0:1]
    return lax.rsqrt(jnp.maximum(deg, 1.0))


def _mm_scale_body(x_ref, w_ref, dout_ref, o_ref):
    ns = _norm_col(dout_ref)
    o_ref[...] = jnp.dot(x_ref[...], w_ref[...],
                         preferred_element_type=jnp.float32) * ns


def _boundary_body(agg_ref, din_ref, dout_ref, b_ref, w_ref, o_ref):
    agg = agg_ref[0] + agg_ref[1]
    nd = _norm_col(din_ref)
    t = jnp.maximum(agg * nd + b_ref[...], 0.0)
    ns = _norm_col(dout_ref)
    o_ref[...] = jnp.dot(t, w_ref[...],
                         preferred_element_type=jnp.float32) * ns


def _final_body(agg_ref, din_ref, b_ref, o_ref):
    agg = agg_ref[0] + agg_ref[1]
    nd = _norm_col(din_ref)
    o_ref[...] = agg * nd + b_ref[...]


def _tc_grid_call(body, n_pad, r, ins, in_specs):
    grid = n_pad // r
    return pl.pallas_call(
        body,
        grid=(grid,),
        in_specs=in_specs,
        out_specs=pl.BlockSpec((r, D), lambda i: (i, 0)),
        out_shape=jax.ShapeDtypeStruct((n_pad, D), jnp.float32),
    )(*ins)


def _spec_rows(r):
    return pl.BlockSpec((r, D), lambda i: (i, 0))


def _spec_deg(r):
    return pl.BlockSpec((NC, r, 16), lambda i: (0, i, 0))


def _spec_agg(r):
    return pl.BlockSpec((NC, r, D), lambda i: (0, i, 0))


def _spec_full(shape):
    nd = len(shape)
    return pl.BlockSpec(shape, lambda i: (0,) * nd)


def kernel(x, edge_index, W1, b1, W2, b2, W3, b3):
    n = x.shape[0]
    e = edge_index.shape[1]

    r = 1024
    n_pad = ((n + 1 + r - 1) // r) * r
    t_chunks = -(-e // (NS * CHUNK))
    if t_chunks % 2:
        t_chunks += 1
    nch0 = max(2, 2 * int(round(t_chunks * 0.4 / 2)))
    nch1 = t_chunks - nch0
    e_pad = NS * t_chunks * CHUNK
    e_per_w = e_pad // NW

    pad_idx = jnp.full((e_pad - e,), n, dtype=jnp.int32)
    src = jnp.concatenate([edge_index[0], pad_idx])
    dst = jnp.concatenate([edge_index[1], pad_idx])
    xp = jnp.pad(x, ((0, n_pad - n), (0, 0)))

    rows_per_tec = n_pad // NS
    zeros_rows = jnp.zeros((rows_per_tec, D), jnp.float32)
    half = D // 2
    col = jnp.arange(D)
    u_src = jnp.broadcast_to((col < half).astype(jnp.float32), (CHUNK, D))
    u_dst = jnp.broadcast_to((col >= half).astype(jnp.float32), (CHUNK, D))

    deg_kernel = _make_deg_kernel(n_pad, e_per_w)
    edge_kernel = _make_edge_kernel(n_pad, nch0, nch1)
    deg_tbl = deg_kernel(src, dst, u_src, u_dst, zeros_rows)
    dsrc = lax.slice(deg_tbl, (0, 0, 0), (NC, n_pad, 16))
    ddst = lax.slice(deg_tbl, (0, 0, half), (NC, n_pad, half + 16))

    b1r = b1.reshape(1, D)
    b2r = b2.reshape(1, D)
    b3r = b3.reshape(1, D)

    h1 = _tc_grid_call(
        _mm_scale_body, n_pad, r,
        [xp, W1, dsrc],
        [_spec_rows(r), _spec_full((D, D)), _spec_deg(r)],
    )
    a1 = edge_kernel(h1, src, dst, zeros_rows)

    h2 = _tc_grid_call(
        _boundary_body, n_pad, r,
        [a1, ddst, dsrc, b1r, W2],
        [_spec_agg(r), _spec_deg(r), _spec_deg(r), _spec_full((1, D)),
         _spec_full((D, D))],
    )
    a2 = edge_kernel(h2, src, dst, zeros_rows)

    h3 = _tc_grid_call(
        _boundary_body, n_pad, r,
        [a2, ddst, dsrc, b2r, W3],
        [_spec_agg(r), _spec_deg(r), _spec_deg(r), _spec_full((1, D)),
         _spec_full((D, D))],
    )
    a3 = edge_kernel(h3, src, dst, zeros_rows)

    out = _tc_grid_call(
        _final_body, n_pad, r,
        [a3, ddst, b3r],
        [_spec_agg(r), _spec_deg(r), _spec_full((1, D))],
    )
    return out[:n]


# speedup vs baseline: 1.6172x; 1.0374x over previous
import functools
import jax
import jax.numpy as jnp
from jax import lax
from jax.experimental import pallas as pl
from jax.experimental.pallas import tpu as pltpu
from jax.experimental.pallas import tpu_sc as plsc

D = 128
CHUNK = 128
NC = 2
NS = 16
NW = NC * NS


def _sc_mesh():
    return plsc.VectorSubcoreMesh(core_axis_name="c", subcore_axis_name="s")


def _make_deg_kernel(n_pad, e_per_w):
    rows_per_tec = n_pad // NS
    n_chunks = e_per_w // CHUNK

    @functools.partial(
        pl.kernel,
        mesh=_sc_mesh(),
        out_type=jax.ShapeDtypeStruct((NC, n_pad, D), jnp.float32),
        scratch_types=[
            pltpu.VMEM((CHUNK,), jnp.int32),
            pltpu.VMEM((CHUNK,), jnp.int32),
            pltpu.VMEM((CHUNK, D), jnp.float32),
            pltpu.VMEM((CHUNK, D), jnp.float32),
            pltpu.VMEM_SHARED((n_pad, D), jnp.float32),
        ],
    )
    def deg_kernel(src_hbm, dst_hbm, usrc_hbm, udst_hbm, zeros_hbm,
                   deg_out,
                   src_v, dst_v, usrc_v, udst_v, deg_sh):
        c = lax.axis_index("c")
        s = lax.axis_index("s")
        w = c * NS + s
        my_rows = pl.ds(s * rows_per_tec, rows_per_tec)
        pltpu.sync_copy(zeros_hbm, deg_sh.at[my_rows])
        pltpu.sync_copy(usrc_hbm, usrc_v)
        pltpu.sync_copy(udst_hbm, udst_v)
        plsc.subcore_barrier()

        def body(i, carry):
            base = pl.multiple_of(w * e_per_w + i * CHUNK, CHUNK)
            pltpu.sync_copy(src_hbm.at[pl.ds(base, CHUNK)], src_v)
            pltpu.sync_copy(dst_hbm.at[pl.ds(base, CHUNK)], dst_v)
            pltpu.sync_copy(usrc_v, deg_sh.at[src_v], add=True)
            pltpu.sync_copy(udst_v, deg_sh.at[dst_v], add=True)
            return carry

        lax.fori_loop(0, n_chunks, body, 0)
        plsc.subcore_barrier()
        pltpu.sync_copy(deg_sh.at[my_rows], deg_out.at[c, my_rows])

    return deg_kernel


def _make_edge_kernel(n_pad, nch0, nch1):
    rows_per_tec = n_pad // NS
    NBUF = 2
    assert nch0 % NBUF == 0 and nch1 % NBUF == 0

    @functools.partial(
        pl.kernel,
        mesh=_sc_mesh(),
        out_type=jax.ShapeDtypeStruct((NC, n_pad, D), jnp.float32),
        scratch_types=[pltpu.VMEM((CHUNK,), jnp.int32)] * (2 * NBUF)
          + [pltpu.VMEM((CHUNK, D), jnp.float32)] * NBUF + [
            pltpu.VMEM_SHARED((n_pad, D), jnp.float32),
        ] + [pltpu.SemaphoreType.DMA] * (3 * NBUF),
    )
    def edge_kernel(h_hbm, src_hbm, dst_hbm, zeros_hbm, agg_out, *rest):
        sidx = rest[:NBUF]
        didx = rest[NBUF:2 * NBUF]
        bufs = rest[2 * NBUF:3 * NBUF]
        agg_sh = rest[3 * NBUF]
        sems = rest[3 * NBUF + 1:]
        gsems = sems[:NBUF]
        isems = sems[NBUF:2 * NBUF]
        dsems = sems[2 * NBUF:]
        c = lax.axis_index("c")
        s = lax.axis_index("s")
        my_rows = pl.ds(s * rows_per_tec, rows_per_tec)
        nch = lax.select(c == 0, nch0, nch1)
        cb = lax.select(c == 0, s * nch0, NS * nch0 + s * nch1)
        pltpu.sync_copy(zeros_hbm, agg_sh.at[my_rows])

        def start_idx(b, chunk):
            base = pl.multiple_of((cb + chunk) * CHUNK, CHUNK)
            pltpu.async_copy(src_hbm.at[pl.ds(base, CHUNK)], sidx[b],
                             isems[b])
            pltpu.async_copy(dst_hbm.at[pl.ds(base, CHUNK)], didx[b],
                             dsems[b])

        def wait_sidx(b):
            pltpu.make_async_copy(src_hbm.at[pl.ds(0, CHUNK)], sidx[b],
                                  isems[b]).wait()

        def wait_didx(b):
            pltpu.make_async_copy(dst_hbm.at[pl.ds(0, CHUNK)], didx[b],
                                  dsems[b]).wait()

        def start_gather(b):
            pltpu.async_copy(h_hbm.at[sidx[b]], bufs[b], gsems[b])

        def wait_gather(b):
            pltpu.make_async_copy(h_hbm.at[sidx[b]], bufs[b],
                                  gsems[b]).wait()

        for b in range(NBUF):
            @pl.when(b < nch)
            def _():
                start_idx(b, b)
                wait_sidx(b)
                start_gather(b)
        plsc.subcore_barrier()

        def body(j, carry):
            for b in range(NBUF):
                cur = j * NBUF + b

                @pl.when(cur < nch)
                def _():
                    wait_gather(b)
                    wait_didx(b)
                    pltpu.sync_copy(bufs[b], agg_sh.at[didx[b]], add=True)
                    nxt = cur + NBUF

                    @pl.when(nxt < nch)
                    def _():
                        start_idx(b, nxt)
                        wait_sidx(b)
                        start_gather(b)
            return carry

        max_nch = max(nch0, nch1)
        lax.fori_loop(0, max_nch // NBUF, body, 0)
        plsc.subcore_barrier()
        pltpu.sync_copy(agg_sh.at[my_rows], agg_out.at[c, my_rows])

    return edge_kernel


def _norm_col(deg_ref):
    deg = deg_ref[0, :, 0:1] + deg_ref[1, :, 0:1]
    return lax.rsqrt(jnp.maximum(deg, 1.0))


def _mm_scale_body(x_ref, w_ref, dout_ref, o_ref):
    ns = _norm_col(dout_ref)
    o_ref[...] = jnp.dot(x_ref[...], w_ref[...],
                         preferred_element_type=jnp.float32) * ns


def _boundary_body(agg_ref, din_ref, dout_ref, b_ref, w_ref, o_ref):
    agg = agg_ref[0] + agg_ref[1]
    nd = _norm_col(din_ref)
    t = jnp.maximum(agg * nd + b_ref[...], 0.0)
    ns = _norm_col(dout_ref)
    o_ref[...] = jnp.dot(t, w_ref[...],
                         preferred_element_type=jnp.float32) * ns


def _final_body(agg_ref, din_ref, b_ref, o_ref):
    agg = agg_ref[0] + agg_ref[1]
    nd = _norm_col(din_ref)
    o_ref[...] = agg * nd + b_ref[...]


def _tc_grid_call(body, n_pad, r, ins, in_specs):
    grid = n_pad // r
    return pl.pallas_call(
        body,
        grid=(grid,),
        in_specs=in_specs,
        out_specs=pl.BlockSpec((r, D), lambda i: (i, 0)),
        out_shape=jax.ShapeDtypeStruct((n_pad, D), jnp.float32),
    )(*ins)


def _spec_rows(r):
    return pl.BlockSpec((r, D), lambda i: (i, 0))


def _spec_deg(r):
    return pl.BlockSpec((NC, r, 16), lambda i: (0, i, 0))


def _spec_agg(r):
    return pl.BlockSpec((NC, r, D), lambda i: (0, i, 0))


def _spec_full(shape):
    nd = len(shape)
    return pl.BlockSpec(shape, lambda i: (0,) * nd)


def kernel(x, edge_index, W1, b1, W2, b2, W3, b3):
    n = x.shape[0]
    e = edge_index.shape[1]

    r = 1024
    n_pad = ((n + 1 + r - 1) // r) * r
    t_chunks = -(-e // (NS * CHUNK))
    if t_chunks % 2:
        t_chunks += 1
    nch0 = max(2, 2 * int(round(t_chunks * 0.5 / 2)))
    nch1 = t_chunks - nch0
    e_pad = NS * t_chunks * CHUNK
    e_per_w = e_pad // NW

    pad_idx = jnp.full((e_pad - e,), n, dtype=jnp.int32)
    src = jnp.concatenate([edge_index[0], pad_idx])
    dst = jnp.concatenate([edge_index[1], pad_idx])
    xp = jnp.pad(x, ((0, n_pad - n), (0, 0)))

    rows_per_tec = n_pad // NS
    zeros_rows = jnp.zeros((rows_per_tec, D), jnp.float32)
    half = D // 2
    col = jnp.arange(D)
    u_src = jnp.broadcast_to((col < half).astype(jnp.float32), (CHUNK, D))
    u_dst = jnp.broadcast_to((col >= half).astype(jnp.float32), (CHUNK, D))

    deg_kernel = _make_deg_kernel(n_pad, e_per_w)
    edge_kernel = _make_edge_kernel(n_pad, nch0, nch1)
    deg_tbl = deg_kernel(src, dst, u_src, u_dst, zeros_rows)
    dsrc = lax.slice(deg_tbl, (0, 0, 0), (NC, n_pad, 16))
    ddst = lax.slice(deg_tbl, (0, 0, half), (NC, n_pad, half + 16))

    b1r = b1.reshape(1, D)
    b2r = b2.reshape(1, D)
    b3r = b3.reshape(1, D)

    h1 = _tc_grid_call(
        _mm_scale_body, n_pad, r,
        [xp, W1, dsrc],
        [_spec_rows(r), _spec_full((D, D)), _spec_deg(r)],
    )
    a1 = edge_kernel(h1, src, dst, zeros_rows)

    h2 = _tc_grid_call(
        _boundary_body, n_pad, r,
        [a1, ddst, dsrc, b1r, W2],
        [_spec_agg(r), _spec_deg(r), _spec_deg(r), _spec_full((1, D)),
         _spec_full((D, D))],
    )
    a2 = edge_kernel(h2, src, dst, zeros_rows)

    h3 = _tc_grid_call(
        _boundary_body, n_pad, r,
        [a2, ddst, dsrc, b2r, W3],
        [_spec_agg(r), _spec_deg(r), _spec_deg(r), _spec_full((1, D)),
         _spec_full((D, D))],
    )
    a3 = edge_kernel(h3, src, dst, zeros_rows)

    out = _tc_grid_call(
        _final_body, n_pad, r,
        [a3, ddst, b3r],
        [_spec_agg(r), _spec_deg(r), _spec_full((1, D))],
    )
    return out[:n]


# speedup vs baseline: 1.8042x; 1.1157x over previous
import functools
import jax
import jax.numpy as jnp
from jax import lax
from jax.experimental import pallas as pl
from jax.experimental.pallas import tpu as pltpu
from jax.experimental.pallas import tpu_sc as plsc

D = 128
CHUNK = 128
NC = 2
NS = 16
NW = NC * NS


def _sc_mesh():
    return plsc.VectorSubcoreMesh(core_axis_name="c", subcore_axis_name="s")


def _make_deg_kernel(n_pad, e_per_w):
    rows_per_tec = n_pad // NS
    n_chunks = e_per_w // CHUNK

    NBUF = 2

    @functools.partial(
        pl.kernel,
        mesh=_sc_mesh(),
        out_type=jax.ShapeDtypeStruct((NC, n_pad, D), jnp.float32),
        scratch_types=[pltpu.VMEM((CHUNK,), jnp.int32)] * (2 * NBUF) + [
            pltpu.VMEM((CHUNK, D), jnp.float32),
            pltpu.VMEM((CHUNK, D), jnp.float32),
            pltpu.VMEM_SHARED((n_pad, D), jnp.float32),
        ] + [pltpu.SemaphoreType.DMA] * (2 * NBUF),
    )
    def deg_kernel(src_hbm, dst_hbm, usrc_hbm, udst_hbm, zeros_hbm,
                   deg_out, *rest):
        sidx = rest[:NBUF]
        didx = rest[NBUF:2 * NBUF]
        usrc_v, udst_v, deg_sh = rest[2 * NBUF:2 * NBUF + 3]
        sems = rest[2 * NBUF + 3:]
        isems = sems[:NBUF]
        dsems = sems[NBUF:]
        c = lax.axis_index("c")
        s = lax.axis_index("s")
        w = c * NS + s
        my_rows = pl.ds(s * rows_per_tec, rows_per_tec)
        pltpu.sync_copy(zeros_hbm, deg_sh.at[my_rows])
        pltpu.sync_copy(usrc_hbm, usrc_v)
        pltpu.sync_copy(udst_hbm, udst_v)

        def start_idx(b, chunk):
            base = pl.multiple_of((w * n_chunks + chunk) * CHUNK, CHUNK)
            pltpu.async_copy(src_hbm.at[pl.ds(base, CHUNK)], sidx[b],
                             isems[b])
            pltpu.async_copy(dst_hbm.at[pl.ds(base, CHUNK)], didx[b],
                             dsems[b])

        def wait_idx(b):
            pltpu.make_async_copy(src_hbm.at[pl.ds(0, CHUNK)], sidx[b],
                                  isems[b]).wait()
            pltpu.make_async_copy(dst_hbm.at[pl.ds(0, CHUNK)], didx[b],
                                  dsems[b]).wait()

        for b in range(NBUF):
            if b < n_chunks:
                start_idx(b, b)
        plsc.subcore_barrier()

        def body(j, carry):
            for b in range(NBUF):
                cur = j * NBUF + b

                @pl.when(cur < n_chunks)
                def _():
                    wait_idx(b)
                    pltpu.sync_copy(usrc_v, deg_sh.at[sidx[b]], add=True)
                    pltpu.sync_copy(udst_v, deg_sh.at[didx[b]], add=True)
                    nxt = cur + NBUF

                    @pl.when(nxt < n_chunks)
                    def _():
                        start_idx(b, nxt)
            return carry

        lax.fori_loop(0, (n_chunks + NBUF - 1) // NBUF, body, 0)
        plsc.subcore_barrier()
        pltpu.sync_copy(deg_sh.at[my_rows], deg_out.at[c, my_rows])

    return deg_kernel


def _make_edge_kernel(n_pad, nch0, nch1):
    rows_per_tec = n_pad // NS
    NBUF = 2
    assert nch0 % NBUF == 0 and nch1 % NBUF == 0

    @functools.partial(
        pl.kernel,
        mesh=_sc_mesh(),
        out_type=jax.ShapeDtypeStruct((NC, n_pad, D), jnp.float32),
        scratch_types=[pltpu.VMEM((CHUNK,), jnp.int32)] * (2 * NBUF)
          + [pltpu.VMEM((CHUNK, D), jnp.float32)] * NBUF + [
            pltpu.VMEM_SHARED((n_pad, D), jnp.float32),
        ] + [pltpu.SemaphoreType.DMA] * (3 * NBUF),
    )
    def edge_kernel(h_hbm, src_hbm, dst_hbm, zeros_hbm, agg_out, *rest):
        sidx = rest[:NBUF]
        didx = rest[NBUF:2 * NBUF]
        bufs = rest[2 * NBUF:3 * NBUF]
        agg_sh = rest[3 * NBUF]
        sems = rest[3 * NBUF + 1:]
        gsems = sems[:NBUF]
        isems = sems[NBUF:2 * NBUF]
        dsems = sems[2 * NBUF:]
        c = lax.axis_index("c")
        s = lax.axis_index("s")
        my_rows = pl.ds(s * rows_per_tec, rows_per_tec)
        nch = lax.select(c == 0, nch0, nch1)
        cb = lax.select(c == 0, s * nch0, NS * nch0 + s * nch1)
        pltpu.sync_copy(zeros_hbm, agg_sh.at[my_rows])

        def start_idx(b, chunk):
            base = pl.multiple_of((cb + chunk) * CHUNK, CHUNK)
            pltpu.async_copy(src_hbm.at[pl.ds(base, CHUNK)], sidx[b],
                             isems[b])
            pltpu.async_copy(dst_hbm.at[pl.ds(base, CHUNK)], didx[b],
                             dsems[b])

        def wait_sidx(b):
            pltpu.make_async_copy(src_hbm.at[pl.ds(0, CHUNK)], sidx[b],
                                  isems[b]).wait()

        def wait_didx(b):
            pltpu.make_async_copy(dst_hbm.at[pl.ds(0, CHUNK)], didx[b],
                                  dsems[b]).wait()

        def start_gather(b):
            pltpu.async_copy(h_hbm.at[sidx[b]], bufs[b], gsems[b])

        def wait_gather(b):
            pltpu.make_async_copy(h_hbm.at[sidx[b]], bufs[b],
                                  gsems[b]).wait()

        for b in range(NBUF):
            @pl.when(b < nch)
            def _():
                start_idx(b, b)
                wait_sidx(b)
                start_gather(b)
        plsc.subcore_barrier()

        def body(j, carry):
            for b in range(NBUF):
                cur = j * NBUF + b

                @pl.when(cur < nch)
                def _():
                    wait_gather(b)
                    wait_didx(b)
                    pltpu.sync_copy(bufs[b], agg_sh.at[didx[b]], add=True)
                    nxt = cur + NBUF

                    @pl.when(nxt < nch)
                    def _():
                        start_idx(b, nxt)
                        wait_sidx(b)
                        start_gather(b)
            return carry

        max_nch = max(nch0, nch1)
        lax.fori_loop(0, max_nch // NBUF, body, 0)
        plsc.subcore_barrier()
        pltpu.sync_copy(agg_sh.at[my_rows], agg_out.at[c, my_rows])

    return edge_kernel


def _norm_col(deg_ref):
    deg = deg_ref[0, :, 0:1] + deg_ref[1, :, 0:1]
    return lax.rsqrt(jnp.maximum(deg, 1.0))


def _mm_scale_body(x_ref, w_ref, dout_ref, o_ref):
    ns = _norm_col(dout_ref)
    o_ref[...] = jnp.dot(x_ref[...], w_ref[...],
                         preferred_element_type=jnp.float32) * ns


def _boundary_body(agg_ref, din_ref, dout_ref, b_ref, w_ref, o_ref):
    agg = agg_ref[0] + agg_ref[1]
    nd = _norm_col(din_ref)
    t = jnp.maximum(agg * nd + b_ref[...], 0.0)
    ns = _norm_col(dout_ref)
    o_ref[...] = jnp.dot(t, w_ref[...],
                         preferred_element_type=jnp.float32) * ns


def _final_body(agg_ref, din_ref, b_ref, o_ref):
    agg = agg_ref[0] + agg_ref[1]
    nd = _norm_col(din_ref)
    o_ref[...] = agg * nd + b_ref[...]


def _tc_grid_call(body, n_pad, r, ins, in_specs):
    grid = n_pad // r
    return pl.pallas_call(
        body,
        grid=(grid,),
        in_specs=in_specs,
        out_specs=pl.BlockSpec((r, D), lambda i: (i, 0)),
        out_shape=jax.ShapeDtypeStruct((n_pad, D), jnp.float32),
    )(*ins)


def _spec_rows(r):
    return pl.BlockSpec((r, D), lambda i: (i, 0))


def _spec_deg(r):
    return pl.BlockSpec((NC, r, 16), lambda i: (0, i, 0))


def _spec_agg(r):
    return pl.BlockSpec((NC, r, D), lambda i: (0, i, 0))


def _spec_full(shape):
    nd = len(shape)
    return pl.BlockSpec(shape, lambda i: (0,) * nd)


def kernel(x, edge_index, W1, b1, W2, b2, W3, b3):
    n = x.shape[0]
    e = edge_index.shape[1]

    r = 1024
    n_pad = ((n + 1 + r - 1) // r) * r
    t_chunks = -(-e // (NS * CHUNK))
    if t_chunks % 2:
        t_chunks += 1
    nch0 = max(2, 2 * int(round(t_chunks * 0.6 / 2)))
    nch1 = t_chunks - nch0
    e_pad = NS * t_chunks * CHUNK
    e_per_w = e_pad // NW

    pad_idx = jnp.full((e_pad - e,), n, dtype=jnp.int32)
    src = jnp.concatenate([edge_index[0], pad_idx])
    dst = jnp.concatenate([edge_index[1], pad_idx])
    xp = jnp.pad(x, ((0, n_pad - n), (0, 0)))

    rows_per_tec = n_pad // NS
    zeros_rows = jnp.zeros((rows_per_tec, D), jnp.float32)
    half = D // 2
    col = jnp.arange(D)
    u_src = jnp.broadcast_to((col < half).astype(jnp.float32), (CHUNK, D))
    u_dst = jnp.broadcast_to((col >= half).astype(jnp.float32), (CHUNK, D))

    deg_kernel = _make_deg_kernel(n_pad, e_per_w)
    edge_kernel = _make_edge_kernel(n_pad, nch0, nch1)
    deg_tbl = deg_kernel(src, dst, u_src, u_dst, zeros_rows)
    dsrc = lax.slice(deg_tbl, (0, 0, 0), (NC, n_pad, 16))
    ddst = lax.slice(deg_tbl, (0, 0, half), (NC, n_pad, half + 16))

    b1r = b1.reshape(1, D)
    b2r = b2.reshape(1, D)
    b3r = b3.reshape(1, D)

    h1 = _tc_grid_call(
        _mm_scale_body, n_pad, r,
        [xp, W1, dsrc],
        [_spec_rows(r), _spec_full((D, D)), _spec_deg(r)],
    )
    a1 = edge_kernel(h1, src, dst, zeros_rows)

    h2 = _tc_grid_call(
        _boundary_body, n_pad, r,
        [a1, ddst, dsrc, b1r, W2],
        [_spec_agg(r), _spec_deg(r), _spec_deg(r), _spec_full((1, D)),
         _spec_full((D, D))],
    )
    a2 = edge_kernel(h2, src, dst, zeros_rows)

    h3 = _tc_grid_call(
        _boundary_body, n_pad, r,
        [a2, ddst, dsrc, b2r, W3],
        [_spec_agg(r), _spec_deg(r), _spec_deg(r), _spec_full((1, D)),
         _spec_full((D, D))],
    )
    a3 = edge_kernel(h3, src, dst, zeros_rows)

    out = _tc_grid_call(
        _final_body, n_pad, r,
        [a3, ddst, b3r],
        [_spec_agg(r), _spec_deg(r), _spec_full((1, D))],
    )
    return out[:n]


# speedup vs baseline: 1.9033x; 1.0549x over previous
import functools
import jax
import jax.numpy as jnp
from jax import lax
from jax.experimental import pallas as pl
from jax.experimental.pallas import tpu as pltpu
from jax.experimental.pallas import tpu_sc as plsc

D = 128
CHUNK = 128
NC = 2
NS = 16
NW = NC * NS


def _sc_mesh():
    return plsc.VectorSubcoreMesh(core_axis_name="c", subcore_axis_name="s")


def _make_deg_kernel(n_pad, e_per_w):
    rows_per_tec = n_pad // NS
    n_chunks = e_per_w // CHUNK

    NBUF = 2

    @functools.partial(
        pl.kernel,
        mesh=_sc_mesh(),
        out_type=jax.ShapeDtypeStruct((NC, n_pad, D), jnp.float32),
        scratch_types=[pltpu.VMEM((CHUNK,), jnp.int32)] * (2 * NBUF) + [
            pltpu.VMEM((CHUNK, D), jnp.float32),
            pltpu.VMEM((CHUNK, D), jnp.float32),
            pltpu.VMEM_SHARED((n_pad, D), jnp.float32),
        ] + [pltpu.SemaphoreType.DMA] * (2 * NBUF),
    )
    def deg_kernel(src_hbm, dst_hbm, usrc_hbm, udst_hbm, zeros_hbm,
                   deg_out, *rest):
        sidx = rest[:NBUF]
        didx = rest[NBUF:2 * NBUF]
        usrc_v, udst_v, deg_sh = rest[2 * NBUF:2 * NBUF + 3]
        sems = rest[2 * NBUF + 3:]
        isems = sems[:NBUF]
        dsems = sems[NBUF:]
        c = lax.axis_index("c")
        s = lax.axis_index("s")
        w = c * NS + s
        my_rows = pl.ds(s * rows_per_tec, rows_per_tec)
        pltpu.sync_copy(zeros_hbm, deg_sh.at[my_rows])
        pltpu.sync_copy(usrc_hbm, usrc_v)
        pltpu.sync_copy(udst_hbm, udst_v)

        def start_idx(b, chunk):
            base = pl.multiple_of((w * n_chunks + chunk) * CHUNK, CHUNK)
            pltpu.async_copy(src_hbm.at[pl.ds(base, CHUNK)], sidx[b],
                             isems[b])
            pltpu.async_copy(dst_hbm.at[pl.ds(base, CHUNK)], didx[b],
                             dsems[b])

        def wait_idx(b):
            pltpu.make_async_copy(src_hbm.at[pl.ds(0, CHUNK)], sidx[b],
                                  isems[b]).wait()
            pltpu.make_async_copy(dst_hbm.at[pl.ds(0, CHUNK)], didx[b],
                                  dsems[b]).wait()

        for b in range(NBUF):
            if b < n_chunks:
                start_idx(b, b)
        plsc.subcore_barrier()

        def body(j, carry):
            for b in range(NBUF):
                cur = j * NBUF + b

                @pl.when(cur < n_chunks)
                def _():
                    wait_idx(b)
                    pltpu.sync_copy(usrc_v, deg_sh.at[sidx[b]], add=True)
                    pltpu.sync_copy(udst_v, deg_sh.at[didx[b]], add=True)
                    nxt = cur + NBUF

                    @pl.when(nxt < n_chunks)
                    def _():
                        start_idx(b, nxt)
            return carry

        lax.fori_loop(0, (n_chunks + NBUF - 1) // NBUF, body, 0)
        plsc.subcore_barrier()
        pltpu.sync_copy(deg_sh.at[my_rows], deg_out.at[c, my_rows])

    return deg_kernel


def _make_edge_kernel(n_pad, nch0, nch1):
    rows_per_tec = n_pad // NS
    NBUF = 2
    assert nch0 % NBUF == 0 and nch1 % NBUF == 0

    @functools.partial(
        pl.kernel,
        mesh=_sc_mesh(),
        out_type=jax.ShapeDtypeStruct((NC, n_pad, D), jnp.float32),
        scratch_types=[pltpu.VMEM((CHUNK,), jnp.int32)] * (2 * NBUF)
          + [pltpu.VMEM((CHUNK, D), jnp.float32)] * NBUF + [
            pltpu.VMEM_SHARED((n_pad, D), jnp.float32),
        ] + [pltpu.SemaphoreType.DMA] * (3 * NBUF),
    )
    def edge_kernel(h_hbm, src_hbm, dst_hbm, zeros_hbm, agg_out, *rest):
        sidx = rest[:NBUF]
        didx = rest[NBUF:2 * NBUF]
        bufs = rest[2 * NBUF:3 * NBUF]
        agg_sh = rest[3 * NBUF]
        sems = rest[3 * NBUF + 1:]
        gsems = sems[:NBUF]
        isems = sems[NBUF:2 * NBUF]
        dsems = sems[2 * NBUF:]
        c = lax.axis_index("c")
        s = lax.axis_index("s")
        my_rows = pl.ds(s * rows_per_tec, rows_per_tec)
        nch = lax.select(c == 0, nch0, nch1)
        cb = lax.select(c == 0, s * nch0, NS * nch0 + s * nch1)
        pltpu.sync_copy(zeros_hbm, agg_sh.at[my_rows])

        def start_idx(b, chunk):
            base = pl.multiple_of((cb + chunk) * CHUNK, CHUNK)
            pltpu.async_copy(src_hbm.at[pl.ds(base, CHUNK)], sidx[b],
                             isems[b])
            pltpu.async_copy(dst_hbm.at[pl.ds(base, CHUNK)], didx[b],
                             dsems[b])

        def wait_sidx(b):
            pltpu.make_async_copy(src_hbm.at[pl.ds(0, CHUNK)], sidx[b],
                                  isems[b]).wait()

        def wait_didx(b):
            pltpu.make_async_copy(dst_hbm.at[pl.ds(0, CHUNK)], didx[b],
                                  dsems[b]).wait()

        def start_gather(b):
            pltpu.async_copy(h_hbm.at[sidx[b]], bufs[b], gsems[b])

        def wait_gather(b):
            pltpu.make_async_copy(h_hbm.at[sidx[b]], bufs[b],
                                  gsems[b]).wait()

        for b in range(NBUF):
            @pl.when(b < nch)
            def _():
                start_idx(b, b)
                wait_sidx(b)
                start_gather(b)
        plsc.subcore_barrier()

        def body(j, carry):
            for b in range(NBUF):
                cur = j * NBUF + b

                @pl.when(cur < nch)
                def _():
                    wait_gather(b)
                    wait_didx(b)
                    pltpu.sync_copy(bufs[b], agg_sh.at[didx[b]], add=True)
                    nxt = cur + NBUF

                    @pl.when(nxt < nch)
                    def _():
                        start_idx(b, nxt)
                        wait_sidx(b)
                        start_gather(b)
            return carry

        max_nch = max(nch0, nch1)
        lax.fori_loop(0, max_nch // NBUF, body, 0)
        plsc.subcore_barrier()
        pltpu.sync_copy(agg_sh.at[my_rows], agg_out.at[c, my_rows])

    return edge_kernel


def _norm_col(deg_ref):
    deg = deg_ref[0, :, 0:1] + deg_ref[1, :, 0:1]
    return lax.rsqrt(jnp.maximum(deg, 1.0))


def _mm_scale_body(x_ref, w_ref, dout_ref, o_ref):
    ns = _norm_col(dout_ref)
    o_ref[...] = jnp.dot(x_ref[...], w_ref[...],
                         preferred_element_type=jnp.float32) * ns


def _boundary_body(agg_ref, din_ref, dout_ref, b_ref, w_ref, o_ref):
    agg = agg_ref[0] + agg_ref[1]
    nd = _norm_col(din_ref)
    t = jnp.maximum(agg * nd + b_ref[...], 0.0)
    ns = _norm_col(dout_ref)
    o_ref[...] = jnp.dot(t, w_ref[...],
                         preferred_element_type=jnp.float32) * ns


def _final_body(agg_ref, din_ref, b_ref, o_ref):
    agg = agg_ref[0] + agg_ref[1]
    nd = _norm_col(din_ref)
    o_ref[...] = agg * nd + b_ref[...]


def _tc_grid_call(body, n_pad, r, ins, in_specs):
    grid = n_pad // r
    return pl.pallas_call(
        body,
        grid=(grid,),
        in_specs=in_specs,
        out_specs=pl.BlockSpec((r, D), lambda i: (i, 0)),
        out_shape=jax.ShapeDtypeStruct((n_pad, D), jnp.float32),
    )(*ins)


def _spec_rows(r):
    return pl.BlockSpec((r, D), lambda i: (i, 0))


def _spec_deg(r):
    return pl.BlockSpec((NC, r, 16), lambda i: (0, i, 0))


def _spec_agg(r):
    return pl.BlockSpec((NC, r, D), lambda i: (0, i, 0))


def _spec_full(shape):
    nd = len(shape)
    return pl.BlockSpec(shape, lambda i: (0,) * nd)


def kernel(x, edge_index, W1, b1, W2, b2, W3, b3):
    n = x.shape[0]
    e = edge_index.shape[1]

    r = 1024
    n_pad = ((n + 1 + r - 1) // r) * r
    t_chunks = -(-e // (NS * CHUNK))
    if t_chunks % 2:
        t_chunks += 1
    nch0 = max(2, 2 * int(round(t_chunks * 0.7 / 2)))
    nch1 = t_chunks - nch0
    e_pad = NS * t_chunks * CHUNK
    e_per_w = e_pad // NW

    pad_idx = jnp.full((e_pad - e,), n, dtype=jnp.int32)
    src = jnp.concatenate([edge_index[0], pad_idx])
    dst = jnp.concatenate([edge_index[1], pad_idx])
    xp = jnp.pad(x, ((0, n_pad - n), (0, 0)))

    rows_per_tec = n_pad // NS
    zeros_rows = jnp.zeros((rows_per_tec, D), jnp.float32)
    half = D // 2
    col = jnp.arange(D)
    u_src = jnp.broadcast_to((col < half).astype(jnp.float32), (CHUNK, D))
    u_dst = jnp.broadcast_to((col >= half).astype(jnp.float32), (CHUNK, D))

    deg_kernel = _make_deg_kernel(n_pad, e_per_w)
    edge_kernel = _make_edge_kernel(n_pad, nch0, nch1)
    deg_tbl = deg_kernel(src, dst, u_src, u_dst, zeros_rows)
    dsrc = lax.slice(deg_tbl, (0, 0, 0), (NC, n_pad, 16))
    ddst = lax.slice(deg_tbl, (0, 0, half), (NC, n_pad, half + 16))

    b1r = b1.reshape(1, D)
    b2r = b2.reshape(1, D)
    b3r = b3.reshape(1, D)

    h1 = _tc_grid_call(
        _mm_scale_body, n_pad, r,
        [xp, W1, dsrc],
        [_spec_rows(r), _spec_full((D, D)), _spec_deg(r)],
    )
    a1 = edge_kernel(h1, src, dst, zeros_rows)

    h2 = _tc_grid_call(
        _boundary_body, n_pad, r,
        [a1, ddst, dsrc, b1r, W2],
        [_spec_agg(r), _spec_deg(r), _spec_deg(r), _spec_full((1, D)),
         _spec_full((D, D))],
    )
    a2 = edge_kernel(h2, src, dst, zeros_rows)

    h3 = _tc_grid_call(
        _boundary_body, n_pad, r,
        [a2, ddst, dsrc, b2r, W3],
        [_spec_agg(r), _spec_deg(r), _spec_deg(r), _spec_full((1, D)),
         _spec_full((D, D))],
    )
    a3 = edge_kernel(h3, src, dst, zeros_rows)

    out = _tc_grid_call(
        _final_body, n_pad, r,
        [a3, ddst, b3r],
        [_spec_agg(r), _spec_deg(r), _spec_full((1, D))],
    )
    return out[:n]


# speedup vs baseline: 2.0216x; 1.0622x over previous
import functools
import jax
import jax.numpy as jnp
from jax import lax
from jax.experimental import pallas as pl
from jax.experimental.pallas import tpu as pltpu
from jax.experimental.pallas import tpu_sc as plsc

D = 128
CHUNK = 128
NC = 2
NS = 16
NW = NC * NS


def _sc_mesh():
    return plsc.VectorSubcoreMesh(core_axis_name="c", subcore_axis_name="s")


def _make_deg_kernel(n_pad, e_per_w):
    rows_per_tec = n_pad // NS
    n_chunks = e_per_w // CHUNK

    NBUF = 2

    @functools.partial(
        pl.kernel,
        mesh=_sc_mesh(),
        out_type=jax.ShapeDtypeStruct((NC, n_pad, D), jnp.float32),
        scratch_types=[pltpu.VMEM((CHUNK,), jnp.int32)] * (2 * NBUF) + [
            pltpu.VMEM((CHUNK, D), jnp.float32),
            pltpu.VMEM((CHUNK, D), jnp.float32),
            pltpu.VMEM_SHARED((n_pad, D), jnp.float32),
        ] + [pltpu.SemaphoreType.DMA] * (2 * NBUF),
    )
    def deg_kernel(src_hbm, dst_hbm, usrc_hbm, udst_hbm, zeros_hbm,
                   deg_out, *rest):
        sidx = rest[:NBUF]
        didx = rest[NBUF:2 * NBUF]
        usrc_v, udst_v, deg_sh = rest[2 * NBUF:2 * NBUF + 3]
        sems = rest[2 * NBUF + 3:]
        isems = sems[:NBUF]
        dsems = sems[NBUF:]
        c = lax.axis_index("c")
        s = lax.axis_index("s")
        w = c * NS + s
        my_rows = pl.ds(s * rows_per_tec, rows_per_tec)
        pltpu.sync_copy(zeros_hbm, deg_sh.at[my_rows])
        pltpu.sync_copy(usrc_hbm, usrc_v)
        pltpu.sync_copy(udst_hbm, udst_v)

        def start_idx(b, chunk):
            base = pl.multiple_of((w * n_chunks + chunk) * CHUNK, CHUNK)
            pltpu.async_copy(src_hbm.at[pl.ds(base, CHUNK)], sidx[b],
                             isems[b])
            pltpu.async_copy(dst_hbm.at[pl.ds(base, CHUNK)], didx[b],
                             dsems[b])

        def wait_idx(b):
            pltpu.make_async_copy(src_hbm.at[pl.ds(0, CHUNK)], sidx[b],
                                  isems[b]).wait()
            pltpu.make_async_copy(dst_hbm.at[pl.ds(0, CHUNK)], didx[b],
                                  dsems[b]).wait()

        for b in range(NBUF):
            if b < n_chunks:
                start_idx(b, b)
        plsc.subcore_barrier()

        def body(j, carry):
            for b in range(NBUF):
                cur = j * NBUF + b

                @pl.when(cur < n_chunks)
                def _():
                    wait_idx(b)
                    pltpu.sync_copy(usrc_v, deg_sh.at[sidx[b]], add=True)
                    pltpu.sync_copy(udst_v, deg_sh.at[didx[b]], add=True)
                    nxt = cur + NBUF

                    @pl.when(nxt < n_chunks)
                    def _():
                        start_idx(b, nxt)
            return carry

        lax.fori_loop(0, (n_chunks + NBUF - 1) // NBUF, body, 0)
        plsc.subcore_barrier()
        pltpu.sync_copy(deg_sh.at[my_rows], deg_out.at[c, my_rows])

    return deg_kernel


def _make_edge_kernel(n_pad, nch0, nch1):
    rows_per_tec = n_pad // NS
    NBUF = 2
    assert nch0 % NBUF == 0 and nch1 % NBUF == 0

    @functools.partial(
        pl.kernel,
        mesh=_sc_mesh(),
        out_type=jax.ShapeDtypeStruct((NC, n_pad, D), jnp.float32),
        scratch_types=[pltpu.VMEM((CHUNK,), jnp.int32)] * (2 * NBUF)
          + [pltpu.VMEM((CHUNK, D), jnp.float32)] * NBUF + [
            pltpu.VMEM_SHARED((n_pad, D), jnp.float32),
        ] + [pltpu.SemaphoreType.DMA] * (3 * NBUF),
    )
    def edge_kernel(h_hbm, src_hbm, dst_hbm, zeros_hbm, agg_out, *rest):
        sidx = rest[:NBUF]
        didx = rest[NBUF:2 * NBUF]
        bufs = rest[2 * NBUF:3 * NBUF]
        agg_sh = rest[3 * NBUF]
        sems = rest[3 * NBUF + 1:]
        gsems = sems[:NBUF]
        isems = sems[NBUF:2 * NBUF]
        dsems = sems[2 * NBUF:]
        c = lax.axis_index("c")
        s = lax.axis_index("s")
        my_rows = pl.ds(s * rows_per_tec, rows_per_tec)
        nch = lax.select(c == 0, nch0, nch1)
        cb = lax.select(c == 0, s * nch0, NS * nch0 + s * nch1)
        pltpu.sync_copy(zeros_hbm, agg_sh.at[my_rows])

        def start_idx(b, chunk):
            base = pl.multiple_of((cb + chunk) * CHUNK, CHUNK)
            pltpu.async_copy(src_hbm.at[pl.ds(base, CHUNK)], sidx[b],
                             isems[b])
            pltpu.async_copy(dst_hbm.at[pl.ds(base, CHUNK)], didx[b],
                             dsems[b])

        def wait_sidx(b):
            pltpu.make_async_copy(src_hbm.at[pl.ds(0, CHUNK)], sidx[b],
                                  isems[b]).wait()

        def wait_didx(b):
            pltpu.make_async_copy(dst_hbm.at[pl.ds(0, CHUNK)], didx[b],
                                  dsems[b]).wait()

        def start_gather(b):
            pltpu.async_copy(h_hbm.at[sidx[b]], bufs[b], gsems[b])

        def wait_gather(b):
            pltpu.make_async_copy(h_hbm.at[sidx[b]], bufs[b],
                                  gsems[b]).wait()

        for b in range(NBUF):
            @pl.when(b < nch)
            def _():
                start_idx(b, b)
                wait_sidx(b)
                start_gather(b)
        plsc.subcore_barrier()

        def body(j, carry):
            for b in range(NBUF):
                cur = j * NBUF + b

                @pl.when(cur < nch)
                def _():
                    wait_gather(b)
                    wait_didx(b)
                    pltpu.sync_copy(bufs[b], agg_sh.at[didx[b]], add=True)
                    nxt = cur + NBUF

                    @pl.when(nxt < nch)
                    def _():
                        start_idx(b, nxt)
                        wait_sidx(b)
                        start_gather(b)
            return carry

        max_nch = max(nch0, nch1)
        lax.fori_loop(0, max_nch // NBUF, body, 0)
        plsc.subcore_barrier()
        pltpu.sync_copy(agg_sh.at[my_rows], agg_out.at[c, my_rows])

    return edge_kernel


def _norm_col(deg_ref):
    deg = deg_ref[0, :, 0:1] + deg_ref[1, :, 0:1]
    return lax.rsqrt(jnp.maximum(deg, 1.0))


def _mm_scale_body(x_ref, w_ref, dout_ref, o_ref):
    ns = _norm_col(dout_ref)
    o_ref[...] = jnp.dot(x_ref[...], w_ref[...],
                         preferred_element_type=jnp.float32) * ns


def _boundary_body(agg_ref, din_ref, dout_ref, b_ref, w_ref, o_ref):
    agg = agg_ref[0] + agg_ref[1]
    nd = _norm_col(din_ref)
    t = jnp.maximum(agg * nd + b_ref[...], 0.0)
    ns = _norm_col(dout_ref)
    o_ref[...] = jnp.dot(t, w_ref[...],
                         preferred_element_type=jnp.float32) * ns


def _final_body(agg_ref, din_ref, b_ref, o_ref):
    agg = agg_ref[0] + agg_ref[1]
    nd = _norm_col(din_ref)
    o_ref[...] = agg * nd + b_ref[...]


def _tc_grid_call(body, n_pad, r, ins, in_specs):
    grid = n_pad // r
    return pl.pallas_call(
        body,
        grid=(grid,),
        in_specs=in_specs,
        out_specs=pl.BlockSpec((r, D), lambda i: (i, 0)),
        out_shape=jax.ShapeDtypeStruct((n_pad, D), jnp.float32),
    )(*ins)


def _spec_rows(r):
    return pl.BlockSpec((r, D), lambda i: (i, 0))


def _spec_deg(r):
    return pl.BlockSpec((NC, r, 16), lambda i: (0, i, 0))


def _spec_agg(r):
    return pl.BlockSpec((NC, r, D), lambda i: (0, i, 0))


def _spec_full(shape):
    nd = len(shape)
    return pl.BlockSpec(shape, lambda i: (0,) * nd)


def kernel(x, edge_index, W1, b1, W2, b2, W3, b3):
    n = x.shape[0]
    e = edge_index.shape[1]

    r = 1024
    n_pad = ((n + 1 + r - 1) // r) * r
    t_chunks = -(-e // (NS * CHUNK))
    if t_chunks % 2:
        t_chunks += 1
    nch0 = max(2, 2 * int(round(t_chunks * 0.8 / 2)))
    nch1 = t_chunks - nch0
    e_pad = NS * t_chunks * CHUNK
    e_per_w = e_pad // NW

    pad_idx = jnp.full((e_pad - e,), n, dtype=jnp.int32)
    src = jnp.concatenate([edge_index[0], pad_idx])
    dst = jnp.concatenate([edge_index[1], pad_idx])
    xp = jnp.pad(x, ((0, n_pad - n), (0, 0)))

    rows_per_tec = n_pad // NS
    zeros_rows = jnp.zeros((rows_per_tec, D), jnp.float32)
    half = D // 2
    col = jnp.arange(D)
    u_src = jnp.broadcast_to((col < half).astype(jnp.float32), (CHUNK, D))
    u_dst = jnp.broadcast_to((col >= half).astype(jnp.float32), (CHUNK, D))

    deg_kernel = _make_deg_kernel(n_pad, e_per_w)
    edge_kernel = _make_edge_kernel(n_pad, nch0, nch1)
    deg_tbl = deg_kernel(src, dst, u_src, u_dst, zeros_rows)
    dsrc = lax.slice(deg_tbl, (0, 0, 0), (NC, n_pad, 16))
    ddst = lax.slice(deg_tbl, (0, 0, half), (NC, n_pad, half + 16))

    b1r = b1.reshape(1, D)
    b2r = b2.reshape(1, D)
    b3r = b3.reshape(1, D)

    h1 = _tc_grid_call(
        _mm_scale_body, n_pad, r,
        [xp, W1, dsrc],
        [_spec_rows(r), _spec_full((D, D)), _spec_deg(r)],
    )
    a1 = edge_kernel(h1, src, dst, zeros_rows)

    h2 = _tc_grid_call(
        _boundary_body, n_pad, r,
        [a1, ddst, dsrc, b1r, W2],
        [_spec_agg(r), _spec_deg(r), _spec_deg(r), _spec_full((1, D)),
         _spec_full((D, D))],
    )
    a2 = edge_kernel(h2, src, dst, zeros_rows)

    h3 = _tc_grid_call(
        _boundary_body, n_pad, r,
        [a2, ddst, dsrc, b2r, W3],
        [_spec_agg(r), _spec_deg(r), _spec_deg(r), _spec_full((1, D)),
         _spec_full((D, D))],
    )
    a3 = edge_kernel(h3, src, dst, zeros_rows)

    out = _tc_grid_call(
        _final_body, n_pad, r,
        [a3, ddst, b3r],
        [_spec_agg(r), _spec_deg(r), _spec_full((1, D))],
    )
    return out[:n]


# speedup vs baseline: 2.1379x; 1.0575x over previous
import functools
import jax
import jax.numpy as jnp
from jax import lax
from jax.experimental import pallas as pl
from jax.experimental.pallas import tpu as pltpu
from jax.experimental.pallas import tpu_sc as plsc

D = 128
CHUNK = 128
NC = 2
NS = 16
NW = NC * NS


def _sc_mesh():
    return plsc.VectorSubcoreMesh(core_axis_name="c", subcore_axis_name="s")


def _make_deg_kernel(n_pad, e_per_w):
    rows_per_tec = n_pad // NS
    n_chunks = e_per_w // CHUNK

    NBUF = 2

    @functools.partial(
        pl.kernel,
        mesh=_sc_mesh(),
        out_type=jax.ShapeDtypeStruct((NC, n_pad, D), jnp.float32),
        scratch_types=[pltpu.VMEM((CHUNK,), jnp.int32)] * (2 * NBUF) + [
            pltpu.VMEM((CHUNK, D), jnp.float32),
            pltpu.VMEM((CHUNK, D), jnp.float32),
            pltpu.VMEM_SHARED((n_pad, D), jnp.float32),
        ] + [pltpu.SemaphoreType.DMA] * (2 * NBUF),
    )
    def deg_kernel(src_hbm, dst_hbm, usrc_hbm, udst_hbm, zeros_hbm,
                   deg_out, *rest):
        sidx = rest[:NBUF]
        didx = rest[NBUF:2 * NBUF]
        usrc_v, udst_v, deg_sh = rest[2 * NBUF:2 * NBUF + 3]
        sems = rest[2 * NBUF + 3:]
        isems = sems[:NBUF]
        dsems = sems[NBUF:]
        c = lax.axis_index("c")
        s = lax.axis_index("s")
        w = c * NS + s
        my_rows = pl.ds(s * rows_per_tec, rows_per_tec)
        pltpu.sync_copy(zeros_hbm, deg_sh.at[my_rows])
        pltpu.sync_copy(usrc_hbm, usrc_v)
        pltpu.sync_copy(udst_hbm, udst_v)

        def start_idx(b, chunk):
            base = pl.multiple_of((w * n_chunks + chunk) * CHUNK, CHUNK)
            pltpu.async_copy(src_hbm.at[pl.ds(base, CHUNK)], sidx[b],
                             isems[b])
            pltpu.async_copy(dst_hbm.at[pl.ds(base, CHUNK)], didx[b],
                             dsems[b])

        def wait_idx(b):
            pltpu.make_async_copy(src_hbm.at[pl.ds(0, CHUNK)], sidx[b],
                                  isems[b]).wait()
            pltpu.make_async_copy(dst_hbm.at[pl.ds(0, CHUNK)], didx[b],
                                  dsems[b]).wait()

        for b in range(NBUF):
            if b < n_chunks:
                start_idx(b, b)
        plsc.subcore_barrier()

        def body(j, carry):
            for b in range(NBUF):
                cur = j * NBUF + b

                @pl.when(cur < n_chunks)
                def _():
                    wait_idx(b)
                    pltpu.sync_copy(usrc_v, deg_sh.at[sidx[b]], add=True)
                    pltpu.sync_copy(udst_v, deg_sh.at[didx[b]], add=True)
                    nxt = cur + NBUF

                    @pl.when(nxt < n_chunks)
                    def _():
                        start_idx(b, nxt)
            return carry

        lax.fori_loop(0, (n_chunks + NBUF - 1) // NBUF, body, 0)
        plsc.subcore_barrier()
        pltpu.sync_copy(deg_sh.at[my_rows], deg_out.at[c, my_rows])

    return deg_kernel


def _make_edge_kernel(n_pad, nch0, nch1):
    rows_per_tec = n_pad // NS
    NBUF = 2
    assert nch0 % NBUF == 0 and nch1 % NBUF == 0

    @functools.partial(
        pl.kernel,
        mesh=_sc_mesh(),
        out_type=jax.ShapeDtypeStruct((NC, n_pad, D), jnp.float32),
        scratch_types=[pltpu.VMEM((CHUNK,), jnp.int32)] * (2 * NBUF)
          + [pltpu.VMEM((CHUNK, D), jnp.float32)] * NBUF + [
            pltpu.VMEM_SHARED((n_pad, D), jnp.float32),
        ] + [pltpu.SemaphoreType.DMA] * (3 * NBUF),
    )
    def edge_kernel(h_hbm, src_hbm, dst_hbm, zeros_hbm, agg_out, *rest):
        sidx = rest[:NBUF]
        didx = rest[NBUF:2 * NBUF]
        bufs = rest[2 * NBUF:3 * NBUF]
        agg_sh = rest[3 * NBUF]
        sems = rest[3 * NBUF + 1:]
        gsems = sems[:NBUF]
        isems = sems[NBUF:2 * NBUF]
        dsems = sems[2 * NBUF:]
        c = lax.axis_index("c")
        s = lax.axis_index("s")
        my_rows = pl.ds(s * rows_per_tec, rows_per_tec)
        nch = lax.select(c == 0, nch0, nch1)
        cb = lax.select(c == 0, s * nch0, NS * nch0 + s * nch1)
        pltpu.sync_copy(zeros_hbm, agg_sh.at[my_rows])

        def start_idx(b, chunk):
            base = pl.multiple_of((cb + chunk) * CHUNK, CHUNK)
            pltpu.async_copy(src_hbm.at[pl.ds(base, CHUNK)], sidx[b],
                             isems[b])
            pltpu.async_copy(dst_hbm.at[pl.ds(base, CHUNK)], didx[b],
                             dsems[b])

        def wait_sidx(b):
            pltpu.make_async_copy(src_hbm.at[pl.ds(0, CHUNK)], sidx[b],
                                  isems[b]).wait()

        def wait_didx(b):
            pltpu.make_async_copy(dst_hbm.at[pl.ds(0, CHUNK)], didx[b],
                                  dsems[b]).wait()

        def start_gather(b):
            pltpu.async_copy(h_hbm.at[sidx[b]], bufs[b], gsems[b])

        def wait_gather(b):
            pltpu.make_async_copy(h_hbm.at[sidx[b]], bufs[b],
                                  gsems[b]).wait()

        for b in range(NBUF):
            @pl.when(b < nch)
            def _():
                start_idx(b, b)
                wait_sidx(b)
                start_gather(b)
        plsc.subcore_barrier()

        def body(j, carry):
            for b in range(NBUF):
                cur = j * NBUF + b

                @pl.when(cur < nch)
                def _():
                    wait_gather(b)
                    wait_didx(b)
                    pltpu.sync_copy(bufs[b], agg_sh.at[didx[b]], add=True)
                    nxt = cur + NBUF

                    @pl.when(nxt < nch)
                    def _():
                        start_idx(b, nxt)
                        wait_sidx(b)
                        start_gather(b)
            return carry

        max_nch = max(nch0, nch1)
        lax.fori_loop(0, max_nch // NBUF, body, 0)
        plsc.subcore_barrier()
        pltpu.sync_copy(agg_sh.at[my_rows], agg_out.at[c, my_rows])

    return edge_kernel


def _norm_col(deg_ref):
    deg = deg_ref[0, :, 0:1] + deg_ref[1, :, 0:1]
    return lax.rsqrt(jnp.maximum(deg, 1.0))


def _mm_scale_body(x_ref, w_ref, dout_ref, o_ref):
    ns = _norm_col(dout_ref)
    o_ref[...] = jnp.dot(x_ref[...], w_ref[...],
                         preferred_element_type=jnp.float32) * ns


def _boundary_body(agg_ref, din_ref, dout_ref, b_ref, w_ref, o_ref):
    agg = agg_ref[0] + agg_ref[1]
    nd = _norm_col(din_ref)
    t = jnp.maximum(agg * nd + b_ref[...], 0.0)
    ns = _norm_col(dout_ref)
    o_ref[...] = jnp.dot(t, w_ref[...],
                         preferred_element_type=jnp.float32) * ns


def _final_body(agg_ref, din_ref, b_ref, o_ref):
    agg = agg_ref[0] + agg_ref[1]
    nd = _norm_col(din_ref)
    o_ref[...] = agg * nd + b_ref[...]


def _tc_grid_call(body, n_pad, r, ins, in_specs):
    grid = n_pad // r
    return pl.pallas_call(
        body,
        grid=(grid,),
        in_specs=in_specs,
        out_specs=pl.BlockSpec((r, D), lambda i: (i, 0)),
        out_shape=jax.ShapeDtypeStruct((n_pad, D), jnp.float32),
    )(*ins)


def _spec_rows(r):
    return pl.BlockSpec((r, D), lambda i: (i, 0))


def _spec_deg(r):
    return pl.BlockSpec((NC, r, 16), lambda i: (0, i, 0))


def _spec_agg(r):
    return pl.BlockSpec((NC, r, D), lambda i: (0, i, 0))


def _spec_full(shape):
    nd = len(shape)
    return pl.BlockSpec(shape, lambda i: (0,) * nd)


def kernel(x, edge_index, W1, b1, W2, b2, W3, b3):
    n = x.shape[0]
    e = edge_index.shape[1]

    r = 1024
    n_pad = ((n + 1 + r - 1) // r) * r
    t_chunks = -(-e // (NS * CHUNK))
    if t_chunks % 2:
        t_chunks += 1
    nch0 = max(2, 2 * int(round(t_chunks * 0.9 / 2)))
    nch1 = t_chunks - nch0
    e_pad = NS * t_chunks * CHUNK
    e_per_w = e_pad // NW

    pad_idx = jnp.full((e_pad - e,), n, dtype=jnp.int32)
    src = jnp.concatenate([edge_index[0], pad_idx])
    dst = jnp.concatenate([edge_index[1], pad_idx])
    xp = jnp.pad(x, ((0, n_pad - n), (0, 0)))

    rows_per_tec = n_pad // NS
    zeros_rows = jnp.zeros((rows_per_tec, D), jnp.float32)
    half = D // 2
    col = jnp.arange(D)
    u_src = jnp.broadcast_to((col < half).astype(jnp.float32), (CHUNK, D))
    u_dst = jnp.broadcast_to((col >= half).astype(jnp.float32), (CHUNK, D))

    deg_kernel = _make_deg_kernel(n_pad, e_per_w)
    edge_kernel = _make_edge_kernel(n_pad, nch0, nch1)
    deg_tbl = deg_kernel(src, dst, u_src, u_dst, zeros_rows)
    dsrc = lax.slice(deg_tbl, (0, 0, 0), (NC, n_pad, 16))
    ddst = lax.slice(deg_tbl, (0, 0, half), (NC, n_pad, half + 16))

    b1r = b1.reshape(1, D)
    b2r = b2.reshape(1, D)
    b3r = b3.reshape(1, D)

    h1 = _tc_grid_call(
        _mm_scale_body, n_pad, r,
        [xp, W1, dsrc],
        [_spec_rows(r), _spec_full((D, D)), _spec_deg(r)],
    )
    a1 = edge_kernel(h1, src, dst, zeros_rows)

    h2 = _tc_grid_call(
        _boundary_body, n_pad, r,
        [a1, ddst, dsrc, b1r, W2],
        [_spec_agg(r), _spec_deg(r), _spec_deg(r), _spec_full((1, D)),
         _spec_full((D, D))],
    )
    a2 = edge_kernel(h2, src, dst, zeros_rows)

    h3 = _tc_grid_call(
        _boundary_body, n_pad, r,
        [a2, ddst, dsrc, b2r, W3],
        [_spec_agg(r), _spec_deg(r), _spec_deg(r), _spec_full((1, D)),
         _spec_full((D, D))],
    )
    a3 = edge_kernel(h3, src, dst, zeros_rows)

    out = _tc_grid_call(
        _final_body, n_pad, r,
        [a3, ddst, b3r],
        [_spec_agg(r), _spec_deg(r), _spec_full((1, D))],
    )
    return out[:n]
